# Initial kernel scaffold; baseline (speedup 1.0000x reference)
import jax
import jax.numpy as jnp
from jax.experimental import pallas as pl


def kernel(x, ref, features_0, features_1, features_2, features_3, aw_w, aw_b, so_w, so_b, ep_w0, ep_b0, ep_w1, ep_b1, ep_w2, ep_b2, ep_w3, ep_b3):
    raise NotImplementedError("write your pallas kernel here")


# speedup vs baseline: 7.9158x; 7.9158x over previous
import functools

import jax
import jax.numpy as jnp
import numpy as np
from jax import lax
from jax.experimental import pallas as pl
from jax.experimental.pallas import tpu as pltpu
from jax.experimental.pallas import tpu_sc as plsc

_NH, _NS = 8, 4
_HEAD_DIM = 32
_B, _L, _P, _DIM = 2, 4, 300, 256
_HW = [128, 64, 32, 16]
_HW2 = [h * h for h in _HW]
_CUM = [0, 16384, 20480, 21504]
_BATCH_ROWS = 21760
_ROWS = _B * _L * _P
_SAMP = _NH * _NS
_NBH = 4 * _HEAD_DIM

_SC_WORKERS = 30
_SC_RPW = 80


def _prep_body(x_ref, ref_ref, awT_ref, awb_ref, soxT_ref, sobx_ref,
               soyT_ref, soby_ref, gmat_ref,
               idx_ref, w00_ref, w01_ref, w10_ref, w11_ref):
    x = x_ref[...]
    a = jnp.dot(x, awT_ref[...], preferred_element_type=jnp.float32)
    a = a + awb_ref[...]
    m = jnp.max(a, axis=1, keepdims=True)
    e = jnp.exp(a - m)
    den = jnp.dot(e, gmat_ref[...], preferred_element_type=jnp.float32)
    attw = e / den

    gx = jnp.tanh(jnp.dot(x, soxT_ref[...], preferred_element_type=jnp.float32)
                  + sobx_ref[...]) + ref_ref[:, 0:1]
    gy = jnp.tanh(jnp.dot(x, soyT_ref[...], preferred_element_type=jnp.float32)
                  + soby_ref[...]) + ref_ref[:, 1:2]

    rows = lax.broadcasted_iota(jnp.int32, (_ROWS, 1), 0)
    li = (rows // _P) % _L
    bi = rows // (_L * _P)
    wm1 = jnp.where(li == 0, _HW[0] - 1,
          jnp.where(li == 1, _HW[1] - 1,
          jnp.where(li == 2, _HW[2] - 1, _HW[3] - 1))).astype(jnp.float32)
    wint = (wm1 + 1.0).astype(jnp.int32)
    base = bi * _BATCH_ROWS + jnp.where(li == 0, _CUM[0],
                              jnp.where(li == 1, _CUM[1],
                              jnp.where(li == 2, _CUM[2], _CUM[3])))

    ix = (gx + 1.0) * 0.5 * wm1
    iy = (gy + 1.0) * 0.5 * wm1
    ix0 = jnp.floor(ix)
    iy0 = jnp.floor(iy)
    wx1 = ix - ix0
    wx0 = 1.0 - wx1
    wy1 = iy - iy0
    wy0 = 1.0 - wy1
    vx0 = ((ix0 >= 0.0) & (ix0 <= wm1)).astype(jnp.float32)
    vx1 = ((ix0 + 1.0 >= 0.0) & (ix0 + 1.0 <= wm1)).astype(jnp.float32)
    vy0 = ((iy0 >= 0.0) & (iy0 <= wm1)).astype(jnp.float32)
    vy1 = ((iy0 + 1.0 >= 0.0) & (iy0 + 1.0 <= wm1)).astype(jnp.float32)
    x0c = jnp.clip(ix0, 0.0, wm1)
    x1c = jnp.clip(ix0 + 1.0, 0.0, wm1)
    y0c = jnp.clip(iy0, 0.0, wm1)
    y1c = jnp.clip(iy0 + 1.0, 0.0, wm1)

    s1x = x1c - x0c
    s1y = y1c - y0c
    xw0 = wx0 * vx0 + (1.0 - s1x) * wx1 * vx1
    xw1 = s1x * wx1 * vx1
    yw0 = wy0 * vy0 + (1.0 - s1y) * wy1 * vy1
    yw1 = s1y * wy1 * vy1

    idx_ref[...] = base + y0c.astype(jnp.int32) * wint + x0c.astype(jnp.int32)
    w00_ref[...] = attw * (yw0 * xw0)
    w01_ref[...] = attw * (yw0 * xw1)
    w10_ref[...] = attw * (yw1 * xw0)
    w11_ref[...] = attw * (yw1 * xw1)


def _prep(x2d, ref2d, awT, awb, soxT, sobx, soyT, soby, gmat):
    shp_i = jax.ShapeDtypeStruct((_ROWS, _SAMP), jnp.int32)
    shp_f = jax.ShapeDtypeStruct((_ROWS, _SAMP), jnp.float32)
    return pl.pallas_call(
        _prep_body,
        out_shape=[shp_i, shp_f, shp_f, shp_f, shp_f],
    )(x2d, ref2d, awT, awb, soxT, sobx, soyT, soby, gmat)


def _proj_body(f_ref, w_ref, o_ref):
    o_ref[0] = lax.dot_general(f_ref[0], w_ref[...],
                               (((0,), (1,)), ((), ())),
                               preferred_element_type=jnp.float32)


def _proj(feat3d, epw, chunk):
    b, c, hw = feat3d.shape
    k = min(chunk, hw)
    return pl.pallas_call(
        _proj_body,
        grid=(b, hw // k),
        in_specs=[pl.BlockSpec((1, c, k), lambda bi, j: (bi, 0, j)),
                  pl.BlockSpec((_HEAD_DIM, c), lambda bi, j: (0, 0))],
        out_specs=pl.BlockSpec((1, k, _HEAD_DIM), lambda bi, j: (bi, j, 0)),
        out_shape=jax.ShapeDtypeStruct((b, hw, _HEAD_DIM), jnp.float32),
    )(feat3d, epw)


def _sc_body(table_hbm, idx_hbm, w_hbm, out_hbm, idx_v, w_v, gbuf, outbuf, sem):
    wid = lax.axis_index("s") * 2 + lax.axis_index("c")

    @pl.when(wid < _SC_WORKERS)
    def _():
        rbase = wid * _SC_RPW
        pltpu.sync_copy(idx_hbm.at[pl.ds(rbase, _SC_RPW)], idx_v)
        pltpu.sync_copy(w_hbm.at[pl.ds(rbase, _SC_RPW)], w_v)

        def body(r, carry):
            pltpu.async_copy(table_hbm.at[idx_v.at[r]], gbuf, sem).wait()
            wrow = [w_v[r, pl.ds(k * 16, 16)] for k in range(4 * _SAMP // 16)]
            for h in range(_NH):
                acc0 = jnp.zeros((16,), jnp.float32)
                acc1 = jnp.zeros((16,), jnp.float32)
                for slot in range(4):
                    for s in range(_NS):
                        pt = h * _NS + s
                        col = slot * _SAMP + pt
                        ws = wrow[col // 16][col % 16]
                        acc0 = acc0 + ws * gbuf[pt, pl.ds(slot * 32, 16)]
                        acc1 = acc1 + ws * gbuf[pt, pl.ds(slot * 32 + 16, 16)]
                outbuf[r, pl.ds(h * _HEAD_DIM, 16)] = acc0
                outbuf[r, pl.ds(h * _HEAD_DIM + 16, 16)] = acc1
            return carry

        lax.fori_loop(0, _SC_RPW, body, 0)
        pltpu.sync_copy(outbuf, out_hbm.at[pl.ds(rbase, _SC_RPW)])


def _sc_gather(table, idx_all, w_all):
    mesh = plsc.VectorSubcoreMesh(core_axis_name="c", subcore_axis_name="s")
    fn = functools.partial(
        pl.kernel,
        mesh=mesh,
        out_type=jax.ShapeDtypeStruct((_ROWS, _NH * _HEAD_DIM), jnp.float32),
        scratch_types=[
            pltpu.VMEM((_SC_RPW, _SAMP), jnp.int32),
            pltpu.VMEM((_SC_RPW, 4 * _SAMP), jnp.float32),
            pltpu.VMEM((_SAMP, _NBH), jnp.float32),
            pltpu.VMEM((_SC_RPW, _NH * _HEAD_DIM), jnp.float32),
            pltpu.SemaphoreType.DMA,
        ],
    )(_sc_body)
    return fn(table, idx_all, w_all)


def _neighborhood(proj, h):
    b = proj.shape[0]
    p4 = proj.reshape(b, h, h, _HEAD_DIM)
    zpad = lambda a, axes: jnp.pad(a, [(0, 1 if i in axes else 0)
                                       for i in range(4)])
    p01 = zpad(p4[:, :, 1:, :], (2,))
    p10 = zpad(p4[:, 1:, :, :], (1,))
    p11 = zpad(p4[:, 1:, 1:, :], (1, 2))
    t = jnp.concatenate([p4, p01, p10, p11], axis=-1)
    return t.reshape(b, h * h, _NBH)


def kernel(x, ref, features_0, features_1, features_2, features_3,
           aw_w, aw_b, so_w, so_b,
           ep_w0, ep_b0, ep_w1, ep_b1, ep_w2, ep_b2, ep_w3, ep_b3):
    b, l, p, dim = x.shape
    x2d = x.reshape(_ROWS, dim)
    ref2d = ref.reshape(_ROWS, 2)

    gmat = jnp.asarray(np.kron(np.eye(_NH, dtype=np.float32),
                               np.ones((_NS, _NS), dtype=np.float32)))
    awT = aw_w.T
    awb = aw_b.reshape(1, _NH * _NS)
    soxT = so_w[0::2].T
    soyT = so_w[1::2].T
    sobx = so_b[0::2].reshape(1, _SAMP)
    soby = so_b[1::2].reshape(1, _SAMP)

    idx, w00, w01, w10, w11 = _prep(
        x2d, ref2d, awT, awb, soxT, sobx, soyT, soby, gmat)
    w_all = jnp.concatenate([w00, w01, w10, w11], axis=1)

    feats = [features_0, features_1, features_2, features_3]
    epws = [ep_w0, ep_w1, ep_w2, ep_w3]
    tables = []
    for i in range(_L):
        f3 = feats[i].reshape(b, feats[i].shape[1], _HW2[i])
        proj = _proj(f3, epws[i], 2048)
        tables.append(_neighborhood(proj, _HW[i]))
    table = jnp.concatenate(
        [tables[i][bi] for bi in range(b) for i in range(_L)], axis=0)

    out2d = _sc_gather(table, idx, w_all)

    epbs = [ep_b0, ep_b1, ep_b2, ep_b3]
    bias = jnp.stack([jnp.tile(eb, _NH) for eb in epbs], axis=0)
    out = out2d.reshape(b, l, p, _NH * _HEAD_DIM) + bias[None, :, None, :]
    return out


# speedup vs baseline: 11.4971x; 1.4524x over previous
import functools

import jax
import jax.numpy as jnp
import numpy as np
from jax import lax
from jax.experimental import pallas as pl
from jax.experimental.pallas import tpu as pltpu
from jax.experimental.pallas import tpu_sc as plsc

_NH, _NS = 8, 4
_HEAD_DIM = 32
_B, _L, _P, _DIM = 2, 4, 300, 256
_HW = [128, 64, 32, 16]
_HW2 = [h * h for h in _HW]
_CUM = [0, 16384, 20480, 21504]
_BATCH_ROWS = 21760
_TABLE_ROWS = _B * _BATCH_ROWS
_ROWS = _B * _L * _P
_SAMP = _NH * _NS
_NBH = 4 * _HEAD_DIM
_TK = 256

_SC_WORKERS = 30
_SC_RPW = 80
_SC_CH = 4
_SC_NCH = _SC_RPW // _SC_CH


def _prep_body(x_ref, ref_ref, awT_ref, awb_ref, soxT_ref, sobx_ref,
               soyT_ref, soby_ref, gmat_ref, idx_ref, w_ref):
    x = x_ref[...]
    a = jnp.dot(x, awT_ref[...], preferred_element_type=jnp.float32)
    a = a + awb_ref[...]
    m = jnp.max(a, axis=1, keepdims=True)
    e = jnp.exp(a - m)
    den = jnp.dot(e, gmat_ref[...], preferred_element_type=jnp.float32)
    attw = e / den

    gx = jnp.tanh(jnp.dot(x, soxT_ref[...], preferred_element_type=jnp.float32)
                  + sobx_ref[...]) + ref_ref[:, 0:1]
    gy = jnp.tanh(jnp.dot(x, soyT_ref[...], preferred_element_type=jnp.float32)
                  + soby_ref[...]) + ref_ref[:, 1:2]

    rows = lax.broadcasted_iota(jnp.int32, (_ROWS, 1), 0)
    li = (rows // _P) % _L
    bi = rows // (_L * _P)
    wm1 = jnp.where(li == 0, _HW[0] - 1,
          jnp.where(li == 1, _HW[1] - 1,
          jnp.where(li == 2, _HW[2] - 1, _HW[3] - 1))).astype(jnp.float32)
    wint = (wm1 + 1.0).astype(jnp.int32)
    base = bi * _BATCH_ROWS + jnp.where(li == 0, _CUM[0],
                              jnp.where(li == 1, _CUM[1],
                              jnp.where(li == 2, _CUM[2], _CUM[3])))

    ix = (gx + 1.0) * 0.5 * wm1
    iy = (gy + 1.0) * 0.5 * wm1
    ix0 = jnp.floor(ix)
    iy0 = jnp.floor(iy)
    wx1 = ix - ix0
    wx0 = 1.0 - wx1
    wy1 = iy - iy0
    wy0 = 1.0 - wy1
    vx0 = ((ix0 >= 0.0) & (ix0 <= wm1)).astype(jnp.float32)
    vx1 = ((ix0 + 1.0 >= 0.0) & (ix0 + 1.0 <= wm1)).astype(jnp.float32)
    vy0 = ((iy0 >= 0.0) & (iy0 <= wm1)).astype(jnp.float32)
    vy1 = ((iy0 + 1.0 >= 0.0) & (iy0 + 1.0 <= wm1)).astype(jnp.float32)
    x0c = jnp.clip(ix0, 0.0, wm1)
    x1c = jnp.clip(ix0 + 1.0, 0.0, wm1)
    y0c = jnp.clip(iy0, 0.0, wm1)
    y1c = jnp.clip(iy0 + 1.0, 0.0, wm1)

    s1x = x1c - x0c
    s1y = y1c - y0c
    xw0 = wx0 * vx0 + (1.0 - s1x) * wx1 * vx1
    xw1 = s1x * wx1 * vx1
    yw0 = wy0 * vy0 + (1.0 - s1y) * wy1 * vy1
    yw1 = s1y * wy1 * vy1

    idx_ref[...] = base + y0c.astype(jnp.int32) * wint + x0c.astype(jnp.int32)
    w_ref[...] = jnp.concatenate(
        [attw * (yw0 * xw0), attw * (yw0 * xw1),
         attw * (yw1 * xw0), attw * (yw1 * xw1)], axis=1)


def _prep(x2d, ref2d, awT, awb, soxT, sobx, soyT, soby, gmat):
    return pl.pallas_call(
        _prep_body,
        out_shape=[jax.ShapeDtypeStruct((_ROWS, _SAMP), jnp.int32),
                   jax.ShapeDtypeStruct((_ROWS, 4 * _SAMP), jnp.float32)],
    )(x2d, ref2d, awT, awb, soxT, sobx, soyT, soby, gmat)


def _table_body(w, fa_ref, fb_ref, epw_ref, *rest):
    o_ref = rest[-1]
    dn = (((0,), (1,)), ((), ()))
    pcur = lax.dot_general(fa_ref[0], epw_ref[...], dn,
                           preferred_element_type=jnp.float32)
    pnxt = lax.dot_general(fb_ref[0], epw_ref[...], dn,
                           preferred_element_type=jnp.float32)[:w]
    last = pl.program_id(1) == pl.num_programs(1) - 1
    pnxt = jnp.where(last, jnp.zeros_like(pnxt), pnxt)
    z1 = jnp.zeros((1, _HEAD_DIM), jnp.float32)
    p01 = jnp.concatenate([pcur[1:], pnxt[0:1]], axis=0)
    p10 = jnp.concatenate([pcur[w:], pnxt], axis=0)
    p11 = jnp.concatenate([pcur[w + 1:], pnxt, z1], axis=0)
    mx = (lax.broadcasted_iota(jnp.int32, (_TK, 1), 0) % w) != (w - 1)
    p01 = jnp.where(mx, p01, 0.0)
    p11 = jnp.where(mx, p11, 0.0)
    o_ref[...] = jnp.concatenate([pcur, p01, p10, p11], axis=1)


def _table_level(feat3d, epw, lvl, table_in):
    b, c, hw = feat3d.shape
    w = _HW[lvl]
    off = _CUM[lvl] // _TK
    boff = _BATCH_ROWS // _TK
    in_specs = [
        pl.BlockSpec((1, c, _TK), lambda bi, j: (bi, 0, j)),
        pl.BlockSpec((1, c, 128), lambda bi, j: (bi, 0, (j + 1) * 2)),
        pl.BlockSpec((_HEAD_DIM, c), lambda bi, j: (0, 0)),
    ]
    args = [feat3d, feat3d, epw]
    alias = {}
    if table_in is not None:
        in_specs.append(pl.BlockSpec(memory_space=pl.ANY))
        args.append(table_in)
        alias = {3: 0}
    return pl.pallas_call(
        functools.partial(_table_body, w),
        grid=(b, hw // _TK),
        in_specs=in_specs,
        out_specs=pl.BlockSpec((_TK, _NBH),
                               lambda bi, j: (bi * boff + off + j, 0)),
        out_shape=jax.ShapeDtypeStruct((_TABLE_ROWS, _NBH), jnp.float32),
        input_output_aliases=alias,
    )(*args)


def _sc_chunk_compute(c, gb, w_v, outbuf):
    def qbody(q, carry):
        r = c * _SC_CH + q
        wrow = [w_v[r, pl.ds(k * 16, 16)] for k in range(4 * _SAMP // 16)]
        for h in range(_NH):
            acc0 = jnp.zeros((16,), jnp.float32)
            acc1 = jnp.zeros((16,), jnp.float32)
            for slot in range(4):
                for s in range(_NS):
                    pt = h * _NS + s
                    col = slot * _SAMP + pt
                    ws = wrow[col // 16][col % 16]
                    gr = q * _SAMP + pt
                    acc0 = acc0 + ws * gb[gr, pl.ds(slot * 32, 16)]
                    acc1 = acc1 + ws * gb[gr, pl.ds(slot * 32 + 16, 16)]
            outbuf[r, pl.ds(h * _HEAD_DIM, 16)] = acc0
            outbuf[r, pl.ds(h * _HEAD_DIM + 16, 16)] = acc1
        return carry
    lax.fori_loop(0, _SC_CH, qbody, 0)


def _sc_body(table_hbm, idx_hbm, w_hbm, out_hbm,
             idx_v, w_v, g0, g1, outbuf, sem0, sem1):
    wid = lax.axis_index("s") * 2 + lax.axis_index("c")

    @pl.when(wid < _SC_WORKERS)
    def _():
        rbase = wid * _SC_RPW
        nrow = _SC_CH * _SAMP
        pltpu.sync_copy(idx_hbm.at[pl.ds(rbase * _SAMP, _SC_RPW * _SAMP)],
                        idx_v)
        pltpu.sync_copy(w_hbm.at[pl.ds(rbase, _SC_RPW)], w_v)

        def start(c, gb, sem):
            o = pl.multiple_of(c * nrow, 8)
            pltpu.async_copy(table_hbm.at[idx_v.at[pl.ds(o, nrow)]], gb, sem)

        def wait(gb, sem):
            pltpu.make_async_copy(table_hbm.at[pl.ds(0, nrow)], gb, sem).wait()

        start(0, g0, sem0)

        def body(g, carry):
            c0 = g * 2
            c1 = c0 + 1
            start(c1, g1, sem1)
            wait(g0, sem0)
            _sc_chunk_compute(c0, g0, w_v, outbuf)

            @pl.when(c1 + 1 < _SC_NCH)
            def _():
                start(c1 + 1, g0, sem0)

            wait(g1, sem1)
            _sc_chunk_compute(c1, g1, w_v, outbuf)
            return carry

        lax.fori_loop(0, _SC_NCH // 2, body, 0)
        pltpu.sync_copy(outbuf, out_hbm.at[pl.ds(rbase, _SC_RPW)])


def _sc_gather(table, idx_flat, w_all):
    mesh = plsc.VectorSubcoreMesh(core_axis_name="c", subcore_axis_name="s")
    fn = functools.partial(
        pl.kernel,
        mesh=mesh,
        out_type=jax.ShapeDtypeStruct((_ROWS, _NH * _HEAD_DIM), jnp.float32),
        scratch_types=[
            pltpu.VMEM((_SC_RPW * _SAMP,), jnp.int32),
            pltpu.VMEM((_SC_RPW, 4 * _SAMP), jnp.float32),
            pltpu.VMEM((_SC_CH * _SAMP, _NBH), jnp.float32),
            pltpu.VMEM((_SC_CH * _SAMP, _NBH), jnp.float32),
            pltpu.VMEM((_SC_RPW, _NH * _HEAD_DIM), jnp.float32),
            pltpu.SemaphoreType.DMA,
            pltpu.SemaphoreType.DMA,
        ],
    )(_sc_body)
    return fn(table, idx_flat, w_all)


def kernel(x, ref, features_0, features_1, features_2, features_3,
           aw_w, aw_b, so_w, so_b,
           ep_w0, ep_b0, ep_w1, ep_b1, ep_w2, ep_b2, ep_w3, ep_b3):
    b, l, p, dim = x.shape
    x2d = x.reshape(_ROWS, dim)
    ref2d = ref.reshape(_ROWS, 2)

    gmat = jnp.asarray(np.kron(np.eye(_NH, dtype=np.float32),
                               np.ones((_NS, _NS), dtype=np.float32)))
    awT = aw_w.T
    awb = aw_b.reshape(1, _NH * _NS)
    soxT = so_w[0::2].T
    soyT = so_w[1::2].T
    sobx = so_b[0::2].reshape(1, _SAMP)
    soby = so_b[1::2].reshape(1, _SAMP)

    idx, w_all = _prep(x2d, ref2d, awT, awb, soxT, sobx, soyT, soby, gmat)

    feats = [features_0, features_1, features_2, features_3]
    epws = [ep_w0, ep_w1, ep_w2, ep_w3]
    table = None
    for i in range(_L):
        f3 = feats[i].reshape(b, feats[i].shape[1], _HW2[i])
        table = _table_level(f3, epws[i], i, table)

    out2d = _sc_gather(table, idx.reshape(-1), w_all)

    epbs = [ep_b0, ep_b1, ep_b2, ep_b3]
    bias = jnp.stack([jnp.tile(eb, _NH) for eb in epbs], axis=0)
    out = out2d.reshape(b, l, p, _NH * _HEAD_DIM) + bias[None, :, None, :]
    return out


# speedup vs baseline: 17.7889x; 1.5473x over previous
import functools

import jax
import jax.numpy as jnp
import numpy as np
from jax import lax
from jax.experimental import pallas as pl
from jax.experimental.pallas import tpu as pltpu
from jax.experimental.pallas import tpu_sc as plsc

_NH, _NS = 8, 4
_HEAD_DIM = 32
_B, _L, _P, _DIM = 2, 4, 300, 256
_HW = [128, 64, 32, 16]
_HW2 = [h * h for h in _HW]
_CUM = [0, 16384, 20480, 22528]
_BATCH_ROWS = 24576
_TABLE_ROWS = _B * _BATCH_ROWS
_TKL = [2048, 2048, 1024, 256]
_ROWS = _B * _L * _P
_SAMP = _NH * _NS
_NBH = 4 * _HEAD_DIM

_SC_WORKERS = 30
_SC_RPW = 80
_SC_CH = 4
_SC_NCH = _SC_RPW // _SC_CH


def _prep_body(x_ref, ref_ref, awT_ref, awb_ref, soxT_ref, sobx_ref,
               soyT_ref, soby_ref, gmat_ref, idx_ref, w_ref):
    x = x_ref[...]
    a = jnp.dot(x, awT_ref[...], preferred_element_type=jnp.float32)
    a = a + awb_ref[...]
    m = jnp.max(a, axis=1, keepdims=True)
    e = jnp.exp(a - m)
    den = jnp.dot(e, gmat_ref[...], preferred_element_type=jnp.float32)
    attw = e / den

    gx = jnp.tanh(jnp.dot(x, soxT_ref[...], preferred_element_type=jnp.float32)
                  + sobx_ref[...]) + ref_ref[:, 0:1]
    gy = jnp.tanh(jnp.dot(x, soyT_ref[...], preferred_element_type=jnp.float32)
                  + soby_ref[...]) + ref_ref[:, 1:2]

    rows = lax.broadcasted_iota(jnp.int32, (_ROWS, 1), 0)
    li = (rows // _P) % _L
    bi = rows // (_L * _P)
    wm1 = jnp.where(li == 0, _HW[0] - 1,
          jnp.where(li == 1, _HW[1] - 1,
          jnp.where(li == 2, _HW[2] - 1, _HW[3] - 1))).astype(jnp.float32)
    wint = (wm1 + 1.0).astype(jnp.int32)
    base = bi * _BATCH_ROWS + jnp.where(li == 0, _CUM[0],
                              jnp.where(li == 1, _CUM[1],
                              jnp.where(li == 2, _CUM[2], _CUM[3])))

    ix = (gx + 1.0) * 0.5 * wm1
    iy = (gy + 1.0) * 0.5 * wm1
    ix0 = jnp.floor(ix)
    iy0 = jnp.floor(iy)
    wx1 = ix - ix0
    wx0 = 1.0 - wx1
    wy1 = iy - iy0
    wy0 = 1.0 - wy1
    vx0 = ((ix0 >= 0.0) & (ix0 <= wm1)).astype(jnp.float32)
    vx1 = ((ix0 + 1.0 >= 0.0) & (ix0 + 1.0 <= wm1)).astype(jnp.float32)
    vy0 = ((iy0 >= 0.0) & (iy0 <= wm1)).astype(jnp.float32)
    vy1 = ((iy0 + 1.0 >= 0.0) & (iy0 + 1.0 <= wm1)).astype(jnp.float32)
    x0c = jnp.clip(ix0, 0.0, wm1)
    x1c = jnp.clip(ix0 + 1.0, 0.0, wm1)
    y0c = jnp.clip(iy0, 0.0, wm1)
    y1c = jnp.clip(iy0 + 1.0, 0.0, wm1)

    s1x = x1c - x0c
    s1y = y1c - y0c
    xw0 = wx0 * vx0 + (1.0 - s1x) * wx1 * vx1
    xw1 = s1x * wx1 * vx1
    yw0 = wy0 * vy0 + (1.0 - s1y) * wy1 * vy1
    yw1 = s1y * wy1 * vy1

    idx_ref[...] = base + y0c.astype(jnp.int32) * wint + x0c.astype(jnp.int32)
    w_ref[...] = jnp.concatenate(
        [attw * (yw0 * xw0), attw * (yw0 * xw1),
         attw * (yw1 * xw0), attw * (yw1 * xw1)], axis=1)


def _prep(x2d, ref2d, awT, awb, soxT, sobx, soyT, soby, gmat):
    return pl.pallas_call(
        _prep_body,
        out_shape=[jax.ShapeDtypeStruct((_ROWS, _SAMP), jnp.int32),
                   jax.ShapeDtypeStruct((_ROWS, 4 * _SAMP), jnp.float32)],
    )(x2d, ref2d, awT, awb, soxT, sobx, soyT, soby, gmat)


def _table_body(w, tk, fa_ref, fb_ref, epw_ref, *rest):
    o_ref = rest[-1]
    dn = (((0,), (1,)), ((), ()))
    pcur = lax.dot_general(fa_ref[0], epw_ref[...], dn,
                           preferred_element_type=jnp.float32)
    pnxt = lax.dot_general(fb_ref[0], epw_ref[...], dn,
                           preferred_element_type=jnp.float32)[:w]
    last = pl.program_id(1) == pl.num_programs(1) - 1
    pnxt = jnp.where(last, jnp.zeros_like(pnxt), pnxt)
    z1 = jnp.zeros((1, _HEAD_DIM), jnp.float32)
    p01 = jnp.concatenate([pcur[1:], pnxt[0:1]], axis=0)
    p10 = jnp.concatenate([pcur[w:], pnxt], axis=0)
    p11 = jnp.concatenate([pcur[w + 1:], pnxt, z1], axis=0)
    mx = (lax.broadcasted_iota(jnp.int32, (tk, 1), 0) % w) != (w - 1)
    p01 = jnp.where(mx, p01, 0.0)
    p11 = jnp.where(mx, p11, 0.0)
    o_ref[...] = jnp.concatenate([pcur, p01, p10, p11], axis=1)


def _table_level(feat3d, epw, lvl, table_in):
    b, c, hw = feat3d.shape
    w = _HW[lvl]
    tk = _TKL[lvl]
    off = _CUM[lvl] // tk
    boff = _BATCH_ROWS // tk
    nb128 = tk // 128
    in_specs = [
        pl.BlockSpec((1, c, tk), lambda bi, j: (bi, 0, j)),
        pl.BlockSpec((1, c, 128), lambda bi, j: (bi, 0, (j + 1) * nb128)),
        pl.BlockSpec((_HEAD_DIM, c), lambda bi, j: (0, 0)),
    ]
    args = [feat3d, feat3d, epw]
    alias = {}
    if table_in is not None:
        in_specs.append(pl.BlockSpec(memory_space=pl.ANY))
        args.append(table_in)
        alias = {3: 0}
    return pl.pallas_call(
        functools.partial(_table_body, w, tk),
        grid=(b, hw // tk),
        in_specs=in_specs,
        out_specs=pl.BlockSpec((tk, _NBH),
                               lambda bi, j: (bi * boff + off + j, 0)),
        out_shape=jax.ShapeDtypeStruct((_TABLE_ROWS, _NBH), jnp.float32),
        input_output_aliases=alias,
    )(*args)


def _sc_chunk_compute(c, gb, w_v, outbuf):
    def qbody(q, carry):
        r = c * _SC_CH + q
        wrow = [w_v[r, pl.ds(k * 16, 16)] for k in range(4 * _SAMP // 16)]
        for h in range(_NH):
            acc0 = jnp.zeros((16,), jnp.float32)
            acc1 = jnp.zeros((16,), jnp.float32)
            for slot in range(4):
                for s in range(_NS):
                    pt = h * _NS + s
                    col = slot * _SAMP + pt
                    ws = wrow[col // 16][col % 16]
                    gr = q * _SAMP + pt
                    acc0 = acc0 + ws * gb[gr, pl.ds(slot * 32, 16)]
                    acc1 = acc1 + ws * gb[gr, pl.ds(slot * 32 + 16, 16)]
            outbuf[r, pl.ds(h * _HEAD_DIM, 16)] = acc0
            outbuf[r, pl.ds(h * _HEAD_DIM + 16, 16)] = acc1
        return carry
    lax.fori_loop(0, _SC_CH, qbody, 0)


def _sc_body(table_hbm, idx_hbm, w_hbm, out_hbm,
             idx_v, w_v, g0, g1, outbuf, sem0, sem1):
    wid = lax.axis_index("s") * 2 + lax.axis_index("c")

    @pl.when(wid < _SC_WORKERS)
    def _():
        rbase = wid * _SC_RPW
        nrow = _SC_CH * _SAMP
        pltpu.sync_copy(idx_hbm.at[pl.ds(rbase * _SAMP, _SC_RPW * _SAMP)],
                        idx_v)
        pltpu.sync_copy(w_hbm.at[pl.ds(rbase, _SC_RPW)], w_v)

        def start(c, gb, sem):
            o = pl.multiple_of(c * nrow, 8)
            pltpu.async_copy(table_hbm.at[idx_v.at[pl.ds(o, nrow)]], gb, sem)

        def wait(gb, sem):
            pltpu.make_async_copy(table_hbm.at[pl.ds(0, nrow)], gb, sem).wait()

        start(0, g0, sem0)

        def body(g, carry):
            c0 = g * 2
            c1 = c0 + 1
            start(c1, g1, sem1)
            wait(g0, sem0)
            _sc_chunk_compute(c0, g0, w_v, outbuf)

            @pl.when(c1 + 1 < _SC_NCH)
            def _():
                start(c1 + 1, g0, sem0)

            wait(g1, sem1)
            _sc_chunk_compute(c1, g1, w_v, outbuf)
            return carry

        lax.fori_loop(0, _SC_NCH // 2, body, 0)
        pltpu.sync_copy(outbuf, out_hbm.at[pl.ds(rbase, _SC_RPW)])


def _sc_gather(table, idx_flat, w_all):
    mesh = plsc.VectorSubcoreMesh(core_axis_name="c", subcore_axis_name="s")
    fn = functools.partial(
        pl.kernel,
        mesh=mesh,
        out_type=jax.ShapeDtypeStruct((_ROWS, _NH * _HEAD_DIM), jnp.float32),
        scratch_types=[
            pltpu.VMEM((_SC_RPW * _SAMP,), jnp.int32),
            pltpu.VMEM((_SC_RPW, 4 * _SAMP), jnp.float32),
            pltpu.VMEM((_SC_CH * _SAMP, _NBH), jnp.float32),
            pltpu.VMEM((_SC_CH * _SAMP, _NBH), jnp.float32),
            pltpu.VMEM((_SC_RPW, _NH * _HEAD_DIM), jnp.float32),
            pltpu.SemaphoreType.DMA,
            pltpu.SemaphoreType.DMA,
        ],
    )(_sc_body)
    return fn(table, idx_flat, w_all)


def kernel(x, ref, features_0, features_1, features_2, features_3,
           aw_w, aw_b, so_w, so_b,
           ep_w0, ep_b0, ep_w1, ep_b1, ep_w2, ep_b2, ep_w3, ep_b3):
    b, l, p, dim = x.shape
    x2d = x.reshape(_ROWS, dim)
    ref2d = ref.reshape(_ROWS, 2)

    gmat = jnp.asarray(np.kron(np.eye(_NH, dtype=np.float32),
                               np.ones((_NS, _NS), dtype=np.float32)))
    awT = aw_w.T
    awb = aw_b.reshape(1, _NH * _NS)
    soxT = so_w[0::2].T
    soyT = so_w[1::2].T
    sobx = so_b[0::2].reshape(1, _SAMP)
    soby = so_b[1::2].reshape(1, _SAMP)

    idx, w_all = _prep(x2d, ref2d, awT, awb, soxT, sobx, soyT, soby, gmat)

    feats = [features_0, features_1, features_2, features_3]
    epws = [ep_w0, ep_w1, ep_w2, ep_w3]
    table = None
    for i in range(_L):
        f3 = feats[i].reshape(b, feats[i].shape[1], _HW2[i])
        table = _table_level(f3, epws[i], i, table)

    out2d = _sc_gather(table, idx.reshape(-1), w_all)

    epbs = [ep_b0, ep_b1, ep_b2, ep_b3]
    bias = jnp.stack([jnp.tile(eb, _NH) for eb in epbs], axis=0)
    out = out2d.reshape(b, l, p, _NH * _HEAD_DIM) + bias[None, :, None, :]
    return out


# speedup vs baseline: 18.2494x; 1.0259x over previous
import functools

import jax
import jax.numpy as jnp
import numpy as np
from jax import lax
from jax.experimental import pallas as pl
from jax.experimental.pallas import tpu as pltpu
from jax.experimental.pallas import tpu_sc as plsc

_NH, _NS = 8, 4
_HEAD_DIM = 32
_B, _L, _P, _DIM = 2, 4, 300, 256
_HW = [128, 64, 32, 16]
_HW2 = [h * h for h in _HW]
_CUM = [0, 16384, 20480, 22528]
_BATCH_ROWS = 24576
_TABLE_ROWS = _B * _BATCH_ROWS
_TKL = [2048, 2048, 1024, 256]
_ROWS = _B * _L * _P
_SAMP = _NH * _NS
_NBH = 4 * _HEAD_DIM

_SC_WORKERS = 30
_SC_RPW = 80
_SC_CH = 4
_SC_NCH = _SC_RPW // _SC_CH


def _prep_body(x_ref, ref_ref, awT_ref, awb_ref, soxT_ref, sobx_ref,
               soyT_ref, soby_ref, gmat_ref, idx_ref, w_ref):
    x = x_ref[...]
    a = jnp.dot(x, awT_ref[...], preferred_element_type=jnp.float32)
    a = a + awb_ref[...]
    m = jnp.max(a, axis=1, keepdims=True)
    e = jnp.exp(a - m)
    den = jnp.dot(e, gmat_ref[...], preferred_element_type=jnp.float32)
    attw = e / den

    gx = jnp.tanh(jnp.dot(x, soxT_ref[...], preferred_element_type=jnp.float32)
                  + sobx_ref[...]) + ref_ref[:, 0:1]
    gy = jnp.tanh(jnp.dot(x, soyT_ref[...], preferred_element_type=jnp.float32)
                  + soby_ref[...]) + ref_ref[:, 1:2]

    rows = lax.broadcasted_iota(jnp.int32, (_ROWS, 1), 0)
    li = (rows // _P) % _L
    bi = rows // (_L * _P)
    wm1 = jnp.where(li == 0, _HW[0] - 1,
          jnp.where(li == 1, _HW[1] - 1,
          jnp.where(li == 2, _HW[2] - 1, _HW[3] - 1))).astype(jnp.float32)
    wint = (wm1 + 1.0).astype(jnp.int32)
    base = bi * _BATCH_ROWS + jnp.where(li == 0, _CUM[0],
                              jnp.where(li == 1, _CUM[1],
                              jnp.where(li == 2, _CUM[2], _CUM[3])))

    ix = (gx + 1.0) * 0.5 * wm1
    iy = (gy + 1.0) * 0.5 * wm1
    ix0 = jnp.floor(ix)
    iy0 = jnp.floor(iy)
    wx1 = ix - ix0
    wx0 = 1.0 - wx1
    wy1 = iy - iy0
    wy0 = 1.0 - wy1
    vx0 = ((ix0 >= 0.0) & (ix0 <= wm1)).astype(jnp.float32)
    vx1 = ((ix0 + 1.0 >= 0.0) & (ix0 + 1.0 <= wm1)).astype(jnp.float32)
    vy0 = ((iy0 >= 0.0) & (iy0 <= wm1)).astype(jnp.float32)
    vy1 = ((iy0 + 1.0 >= 0.0) & (iy0 + 1.0 <= wm1)).astype(jnp.float32)
    x0c = jnp.clip(ix0, 0.0, wm1)
    x1c = jnp.clip(ix0 + 1.0, 0.0, wm1)
    y0c = jnp.clip(iy0, 0.0, wm1)
    y1c = jnp.clip(iy0 + 1.0, 0.0, wm1)

    s1x = x1c - x0c
    s1y = y1c - y0c
    xw0 = wx0 * vx0 + (1.0 - s1x) * wx1 * vx1
    xw1 = s1x * wx1 * vx1
    yw0 = wy0 * vy0 + (1.0 - s1y) * wy1 * vy1
    yw1 = s1y * wy1 * vy1

    idx_ref[...] = base + y0c.astype(jnp.int32) * wint + x0c.astype(jnp.int32)
    w_ref[...] = jnp.concatenate(
        [attw * (yw0 * xw0), attw * (yw0 * xw1),
         attw * (yw1 * xw0), attw * (yw1 * xw1)], axis=1)


def _prep(x2d, ref2d, awT, awb, soxT, sobx, soyT, soby, gmat):
    return pl.pallas_call(
        _prep_body,
        out_shape=[jax.ShapeDtypeStruct((_ROWS, _SAMP), jnp.int32),
                   jax.ShapeDtypeStruct((_ROWS, 4 * _SAMP), jnp.float32)],
    )(x2d, ref2d, awT, awb, soxT, sobx, soyT, soby, gmat)


def _table_body(w, tk, fa_ref, fb_ref, epw_ref, *rest):
    o_ref = rest[-1]
    dn = (((0,), (1,)), ((), ()))
    pcur = lax.dot_general(fa_ref[0], epw_ref[...], dn,
                           preferred_element_type=jnp.float32)
    pnxt = lax.dot_general(fb_ref[0], epw_ref[...], dn,
                           preferred_element_type=jnp.float32)[:w]
    last = pl.program_id(1) == pl.num_programs(1) - 1
    pnxt = jnp.where(last, jnp.zeros_like(pnxt), pnxt)
    z1 = jnp.zeros((1, _HEAD_DIM), jnp.float32)
    p01 = jnp.concatenate([pcur[1:], pnxt[0:1]], axis=0)
    p10 = jnp.concatenate([pcur[w:], pnxt], axis=0)
    p11 = jnp.concatenate([pcur[w + 1:], pnxt, z1], axis=0)
    mx = (lax.broadcasted_iota(jnp.int32, (tk, 1), 0) % w) != (w - 1)
    p01 = jnp.where(mx, p01, 0.0)
    p11 = jnp.where(mx, p11, 0.0)
    o_ref[...] = jnp.concatenate([pcur, p01, p10, p11], axis=1)


def _table_level(feat3d, epw, lvl, table_in):
    b, c, hw = feat3d.shape
    w = _HW[lvl]
    tk = _TKL[lvl]
    off = _CUM[lvl] // tk
    boff = _BATCH_ROWS // tk
    nb128 = tk // 128
    in_specs = [
        pl.BlockSpec((1, c, tk), lambda bi, j: (bi, 0, j)),
        pl.BlockSpec((1, c, 128), lambda bi, j: (bi, 0, (j + 1) * nb128)),
        pl.BlockSpec((_HEAD_DIM, c), lambda bi, j: (0, 0)),
    ]
    args = [feat3d, feat3d, epw]
    alias = {}
    if table_in is not None:
        in_specs.append(pl.BlockSpec(memory_space=pl.ANY))
        args.append(table_in)
        alias = {3: 0}
    return pl.pallas_call(
        functools.partial(_table_body, w, tk),
        grid=(b, hw // tk),
        in_specs=in_specs,
        out_specs=pl.BlockSpec((tk, _NBH),
                               lambda bi, j: (bi * boff + off + j, 0)),
        out_shape=jax.ShapeDtypeStruct((_TABLE_ROWS, _NBH), jnp.float32),
        input_output_aliases=alias,
    )(*args)


def _sc_chunk_compute(c, gb, w_v, outbuf):
    def qbody(q, carry):
        r = c * _SC_CH + q
        wrow = [w_v[r, pl.ds(k * 16, 16)] for k in range(4 * _SAMP // 16)]
        for h in range(_NH):
            acc0 = jnp.zeros((16,), jnp.float32)
            acc1 = jnp.zeros((16,), jnp.float32)
            for slot in range(4):
                for s in range(_NS):
                    pt = h * _NS + s
                    col = slot * _SAMP + pt
                    ws = wrow[col // 16][col % 16]
                    gr = q * _SAMP + pt
                    acc0 = acc0 + ws * gb[gr, pl.ds(slot * 32, 16)]
                    acc1 = acc1 + ws * gb[gr, pl.ds(slot * 32 + 16, 16)]
            outbuf[r, pl.ds(h * _HEAD_DIM, 16)] = acc0
            outbuf[r, pl.ds(h * _HEAD_DIM + 16, 16)] = acc1
        return carry
    lax.fori_loop(0, _SC_CH, qbody, 0)


def _sc_body(table_hbm, idx_hbm, w_hbm, out_hbm,
             idx_v, w_v, g0, g1, g2, g3, outbuf, sem0, sem1, sem2, sem3):
    wid = lax.axis_index("s") * 2 + lax.axis_index("c")
    gbs = [g0, g1, g2, g3]
    sems = [sem0, sem1, sem2, sem3]

    @pl.when(wid < _SC_WORKERS)
    def _():
        rbase = wid * _SC_RPW
        nrow = _SC_CH * _SAMP
        pltpu.sync_copy(idx_hbm.at[pl.ds(rbase * _SAMP, _SC_RPW * _SAMP)],
                        idx_v)
        pltpu.sync_copy(w_hbm.at[pl.ds(rbase, _SC_RPW)], w_v)

        def start(c, k):
            o = pl.multiple_of(c * nrow, 8)
            pltpu.async_copy(table_hbm.at[idx_v.at[pl.ds(o, nrow)]],
                             gbs[k], sems[k])

        def wait(k):
            pltpu.make_async_copy(table_hbm.at[pl.ds(0, nrow)],
                                  gbs[k], sems[k]).wait()

        for k in range(4):
            start(k, k)

        def body(g, carry):
            c0 = g * 4
            for k in range(4):
                c = c0 + k
                wait(k)
                _sc_chunk_compute(c, gbs[k], w_v, outbuf)

                @pl.when(c + 4 < _SC_NCH)
                def _():
                    start(c + 4, k)
            return carry

        lax.fori_loop(0, _SC_NCH // 4, body, 0)
        pltpu.sync_copy(outbuf, out_hbm.at[pl.ds(rbase, _SC_RPW)])


def _sc_gather(table, idx_flat, w_all):
    mesh = plsc.VectorSubcoreMesh(core_axis_name="c", subcore_axis_name="s")
    gshape = pltpu.VMEM((_SC_CH * _SAMP, _NBH), jnp.float32)
    fn = functools.partial(
        pl.kernel,
        mesh=mesh,
        out_type=jax.ShapeDtypeStruct((_ROWS, _NH * _HEAD_DIM), jnp.float32),
        scratch_types=[
            pltpu.VMEM((_SC_RPW * _SAMP,), jnp.int32),
            pltpu.VMEM((_SC_RPW, 4 * _SAMP), jnp.float32),
            gshape, gshape, gshape, gshape,
            pltpu.VMEM((_SC_RPW, _NH * _HEAD_DIM), jnp.float32),
            pltpu.SemaphoreType.DMA,
            pltpu.SemaphoreType.DMA,
            pltpu.SemaphoreType.DMA,
            pltpu.SemaphoreType.DMA,
        ],
    )(_sc_body)
    return fn(table, idx_flat, w_all)


def kernel(x, ref, features_0, features_1, features_2, features_3,
           aw_w, aw_b, so_w, so_b,
           ep_w0, ep_b0, ep_w1, ep_b1, ep_w2, ep_b2, ep_w3, ep_b3):
    b, l, p, dim = x.shape
    x2d = x.reshape(_ROWS, dim)
    ref2d = ref.reshape(_ROWS, 2)

    gmat = jnp.asarray(np.kron(np.eye(_NH, dtype=np.float32),
                               np.ones((_NS, _NS), dtype=np.float32)))
    awT = aw_w.T
    awb = aw_b.reshape(1, _NH * _NS)
    soxT = so_w[0::2].T
    soyT = so_w[1::2].T
    sobx = so_b[0::2].reshape(1, _SAMP)
    soby = so_b[1::2].reshape(1, _SAMP)

    idx, w_all = _prep(x2d, ref2d, awT, awb, soxT, sobx, soyT, soby, gmat)

    feats = [features_0, features_1, features_2, features_3]
    epws = [ep_w0, ep_w1, ep_w2, ep_w3]
    table = None
    for i in range(_L):
        f3 = feats[i].reshape(b, feats[i].shape[1], _HW2[i])
        table = _table_level(f3, epws[i], i, table)

    out2d = _sc_gather(table, idx.reshape(-1), w_all)

    epbs = [ep_b0, ep_b1, ep_b2, ep_b3]
    bias = jnp.stack([jnp.tile(eb, _NH) for eb in epbs], axis=0)
    out = out2d.reshape(b, l, p, _NH * _HEAD_DIM) + bias[None, :, None, :]
    return out


# speedup vs baseline: 18.6475x; 1.0218x over previous
import functools

import jax
import jax.numpy as jnp
import numpy as np
from jax import lax
from jax.experimental import pallas as pl
from jax.experimental.pallas import tpu as pltpu
from jax.experimental.pallas import tpu_sc as plsc

_NH, _NS = 8, 4
_HEAD_DIM = 32
_B, _L, _P, _DIM = 2, 4, 300, 256
_HW = [128, 64, 32, 16]
_HW2 = [h * h for h in _HW]
_CUM = [0, 16384, 20480, 22528]
_BATCH_ROWS = 24576
_TABLE_ROWS = _B * _BATCH_ROWS
_TKL = [2048, 2048, 1024, 256]
_ROWS = _B * _L * _P
_SAMP = _NH * _NS
_NBH = 4 * _HEAD_DIM

_SC_WORKERS = 30
_SC_RPW = 80
_SC_CH = 4
_SC_NCH = _SC_RPW // _SC_CH


def _prep_body(x_ref, ref_ref, awT_ref, awb_ref, soxT_ref, sobx_ref,
               soyT_ref, soby_ref, gmat_ref, idx_ref, w_ref):
    x = x_ref[...]
    a = jnp.dot(x, awT_ref[...], preferred_element_type=jnp.float32)
    a = a + awb_ref[...]
    m = jnp.max(a, axis=1, keepdims=True)
    e = jnp.exp(a - m)
    den = jnp.dot(e, gmat_ref[...], preferred_element_type=jnp.float32)
    attw = e / den

    gx = jnp.tanh(jnp.dot(x, soxT_ref[...], preferred_element_type=jnp.float32)
                  + sobx_ref[...]) + ref_ref[:, 0:1]
    gy = jnp.tanh(jnp.dot(x, soyT_ref[...], preferred_element_type=jnp.float32)
                  + soby_ref[...]) + ref_ref[:, 1:2]

    rows = lax.broadcasted_iota(jnp.int32, (_ROWS, 1), 0)
    li = (rows // _P) % _L
    bi = rows // (_L * _P)
    wm1 = jnp.where(li == 0, _HW[0] - 1,
          jnp.where(li == 1, _HW[1] - 1,
          jnp.where(li == 2, _HW[2] - 1, _HW[3] - 1))).astype(jnp.float32)
    wint = (wm1 + 1.0).astype(jnp.int32)
    base = bi * _BATCH_ROWS + jnp.where(li == 0, _CUM[0],
                              jnp.where(li == 1, _CUM[1],
                              jnp.where(li == 2, _CUM[2], _CUM[3])))

    ix = (gx + 1.0) * 0.5 * wm1
    iy = (gy + 1.0) * 0.5 * wm1
    ix0 = jnp.floor(ix)
    iy0 = jnp.floor(iy)
    wx1 = ix - ix0
    wx0 = 1.0 - wx1
    wy1 = iy - iy0
    wy0 = 1.0 - wy1
    vx0 = ((ix0 >= 0.0) & (ix0 <= wm1)).astype(jnp.float32)
    vx1 = ((ix0 + 1.0 >= 0.0) & (ix0 + 1.0 <= wm1)).astype(jnp.float32)
    vy0 = ((iy0 >= 0.0) & (iy0 <= wm1)).astype(jnp.float32)
    vy1 = ((iy0 + 1.0 >= 0.0) & (iy0 + 1.0 <= wm1)).astype(jnp.float32)
    x0c = jnp.clip(ix0, 0.0, wm1)
    x1c = jnp.clip(ix0 + 1.0, 0.0, wm1)
    y0c = jnp.clip(iy0, 0.0, wm1)
    y1c = jnp.clip(iy0 + 1.0, 0.0, wm1)

    s1x = x1c - x0c
    s1y = y1c - y0c
    xw0 = wx0 * vx0 + (1.0 - s1x) * wx1 * vx1
    xw1 = s1x * wx1 * vx1
    yw0 = wy0 * vy0 + (1.0 - s1y) * wy1 * vy1
    yw1 = s1y * wy1 * vy1

    idx_ref[...] = base + y0c.astype(jnp.int32) * wint + x0c.astype(jnp.int32)
    w_ref[...] = jnp.concatenate(
        [attw * (yw0 * xw0), attw * (yw0 * xw1),
         attw * (yw1 * xw0), attw * (yw1 * xw1)], axis=1)


def _level_part(fa_ref, fb_ref, epw_ref, tbl_ref, w, rows, last):
    dn = (((0,), (1,)), ((), ()))
    pcur = lax.dot_general(fa_ref[0][:, :rows], epw_ref[...], dn,
                           preferred_element_type=jnp.float32)
    pnxt = lax.dot_general(fb_ref[0], epw_ref[...], dn,
                           preferred_element_type=jnp.float32)[:w]
    pnxt = jnp.where(last, jnp.zeros_like(pnxt), pnxt)
    z1 = jnp.zeros((1, _HEAD_DIM), jnp.float32)
    p01 = jnp.concatenate([pcur[1:], pnxt[0:1]], axis=0)
    p10 = jnp.concatenate([pcur[w:], pnxt], axis=0)
    p11 = jnp.concatenate([pcur[w + 1:], pnxt, z1], axis=0)
    mx = (lax.broadcasted_iota(jnp.int32, (rows, 1), 0) % w) != (w - 1)
    p01 = jnp.where(mx, p01, 0.0)
    p11 = jnp.where(mx, p11, 0.0)
    tbl_ref[0:rows, :] = jnp.concatenate([pcur, p01, p10, p11], axis=1)


def _fused_body(x_ref, ref_ref, awT_ref, awb_ref, soxT_ref, sobx_ref,
                soyT_ref, soby_ref, gmat_ref,
                f0a_ref, f0b_ref, f1a_ref, f1b_ref,
                f2a_ref, f2b_ref, f3a_ref, f3b_ref,
                w0_ref, w1_ref, w2_ref, w3_ref,
                idx_ref, w_ref, tbl_ref):
    bi = pl.program_id(0)
    j = pl.program_id(1)

    @pl.when((bi == 0) & (j == 0))
    def _():
        _prep_body(x_ref, ref_ref, awT_ref, awb_ref, soxT_ref, sobx_ref,
                   soyT_ref, soby_ref, gmat_ref, idx_ref, w_ref)

    @pl.when(j < 8)
    def _():
        _level_part(f0a_ref, f0b_ref, w0_ref, tbl_ref, _HW[0], 2048, j == 7)

    @pl.when((j == 8) | (j == 9))
    def _():
        _level_part(f1a_ref, f1b_ref, w1_ref, tbl_ref, _HW[1], 2048, j == 9)

    @pl.when(j == 10)
    def _():
        _level_part(f2a_ref, f2b_ref, w2_ref, tbl_ref, _HW[2], 1024, True)

    @pl.when(j == 11)
    def _():
        _level_part(f3a_ref, f3b_ref, w3_ref, tbl_ref, _HW[3], 256, True)


def _fused_tc(x2d, ref2d, awT, awb, soxT, sobx, soyT, soby, gmat,
              feats, epws):
    cs = [f.shape[1] for f in feats]
    full = lambda c: pl.BlockSpec((_HEAD_DIM, c), lambda bi, j: (0, 0))
    one = lambda shp: pl.BlockSpec(shp, lambda bi, j: (0, 0))
    in_specs = [
        one((_ROWS, _DIM)), one((_ROWS, 2)),
        one((_DIM, _SAMP)), one((1, _SAMP)),
        one((_DIM, _SAMP)), one((1, _SAMP)),
        one((_DIM, _SAMP)), one((1, _SAMP)),
        one((_SAMP, _SAMP)),
        pl.BlockSpec((1, cs[0], 2048),
                     lambda bi, j: (bi, 0, jnp.minimum(j, 7))),
        pl.BlockSpec((1, cs[0], 128),
                     lambda bi, j: (bi, 0, (jnp.minimum(j, 7) + 1) * 16)),
        pl.BlockSpec((1, cs[1], 2048),
                     lambda bi, j: (bi, 0, jnp.clip(j - 8, 0, 1))),
        pl.BlockSpec((1, cs[1], 128),
                     lambda bi, j: (bi, 0, (jnp.clip(j - 8, 0, 1) + 1) * 16)),
        pl.BlockSpec((1, cs[2], 1024), lambda bi, j: (bi, 0, 0)),
        pl.BlockSpec((1, cs[2], 128), lambda bi, j: (bi, 0, 7)),
        pl.BlockSpec((1, cs[3], 256), lambda bi, j: (bi, 0, 0)),
        pl.BlockSpec((1, cs[3], 128), lambda bi, j: (bi, 0, 1)),
        full(cs[0]), full(cs[1]), full(cs[2]), full(cs[3]),
    ]
    out_specs = [
        pl.BlockSpec((_ROWS, _SAMP), lambda bi, j: (0, 0)),
        pl.BlockSpec((_ROWS, 4 * _SAMP), lambda bi, j: (0, 0)),
        pl.BlockSpec((2048, _NBH), lambda bi, j: (bi * 12 + j, 0)),
    ]
    return pl.pallas_call(
        _fused_body,
        grid=(_B, 12),
        in_specs=in_specs,
        out_specs=out_specs,
        out_shape=[jax.ShapeDtypeStruct((_ROWS, _SAMP), jnp.int32),
                   jax.ShapeDtypeStruct((_ROWS, 4 * _SAMP), jnp.float32),
                   jax.ShapeDtypeStruct((_TABLE_ROWS, _NBH), jnp.float32)],
    )(x2d, ref2d, awT, awb, soxT, sobx, soyT, soby, gmat,
      feats[0], feats[0], feats[1], feats[1],
      feats[2], feats[2], feats[3], feats[3],
      epws[0], epws[1], epws[2], epws[3])


def _sc_chunk_compute(c, gb, w_v, outbuf):
    def qbody(q, carry):
        r = c * _SC_CH + q
        wrow = [w_v[r, pl.ds(k * 16, 16)] for k in range(4 * _SAMP // 16)]
        for h in range(_NH):
            acc0 = jnp.zeros((16,), jnp.float32)
            acc1 = jnp.zeros((16,), jnp.float32)
            for slot in range(4):
                for s in range(_NS):
                    pt = h * _NS + s
                    col = slot * _SAMP + pt
                    ws = wrow[col // 16][col % 16]
                    gr = q * _SAMP + pt
                    acc0 = acc0 + ws * gb[gr, pl.ds(slot * 32, 16)]
                    acc1 = acc1 + ws * gb[gr, pl.ds(slot * 32 + 16, 16)]
            outbuf[r, pl.ds(h * _HEAD_DIM, 16)] = acc0
            outbuf[r, pl.ds(h * _HEAD_DIM + 16, 16)] = acc1
        return carry
    lax.fori_loop(0, _SC_CH, qbody, 0)


def _sc_body(table_hbm, idx_hbm, w_hbm, out_hbm,
             idx_v, w_v, g0, g1, g2, g3, outbuf, sem0, sem1, sem2, sem3):
    wid = lax.axis_index("s") * 2 + lax.axis_index("c")
    gbs = [g0, g1, g2, g3]
    sems = [sem0, sem1, sem2, sem3]

    @pl.when(wid < _SC_WORKERS)
    def _():
        rbase = wid * _SC_RPW
        nrow = _SC_CH * _SAMP
        pltpu.sync_copy(idx_hbm.at[pl.ds(rbase * _SAMP, _SC_RPW * _SAMP)],
                        idx_v)
        pltpu.sync_copy(w_hbm.at[pl.ds(rbase, _SC_RPW)], w_v)

        def start(c, k):
            o = pl.multiple_of(c * nrow, 8)
            pltpu.async_copy(table_hbm.at[idx_v.at[pl.ds(o, nrow)]],
                             gbs[k], sems[k])

        def wait(k):
            pltpu.make_async_copy(table_hbm.at[pl.ds(0, nrow)],
                                  gbs[k], sems[k]).wait()

        for k in range(4):
            start(k, k)

        def body(g, carry):
            c0 = g * 4
            for k in range(4):
                c = c0 + k
                wait(k)
                _sc_chunk_compute(c, gbs[k], w_v, outbuf)

                @pl.when(c + 4 < _SC_NCH)
                def _():
                    start(c + 4, k)
            return carry

        lax.fori_loop(0, _SC_NCH // 4, body, 0)
        pltpu.sync_copy(outbuf, out_hbm.at[pl.ds(rbase, _SC_RPW)])


def _sc_gather(table, idx_flat, w_all):
    mesh = plsc.VectorSubcoreMesh(core_axis_name="c", subcore_axis_name="s")
    gshape = pltpu.VMEM((_SC_CH * _SAMP, _NBH), jnp.float32)
    fn = functools.partial(
        pl.kernel,
        mesh=mesh,
        out_type=jax.ShapeDtypeStruct((_ROWS, _NH * _HEAD_DIM), jnp.float32),
        scratch_types=[
            pltpu.VMEM((_SC_RPW * _SAMP,), jnp.int32),
            pltpu.VMEM((_SC_RPW, 4 * _SAMP), jnp.float32),
            gshape, gshape, gshape, gshape,
            pltpu.VMEM((_SC_RPW, _NH * _HEAD_DIM), jnp.float32),
            pltpu.SemaphoreType.DMA,
            pltpu.SemaphoreType.DMA,
            pltpu.SemaphoreType.DMA,
            pltpu.SemaphoreType.DMA,
        ],
    )(_sc_body)
    return fn(table, idx_flat, w_all)


def kernel(x, ref, features_0, features_1, features_2, features_3,
           aw_w, aw_b, so_w, so_b,
           ep_w0, ep_b0, ep_w1, ep_b1, ep_w2, ep_b2, ep_w3, ep_b3):
    b, l, p, dim = x.shape
    x2d = x.reshape(_ROWS, dim)
    ref2d = ref.reshape(_ROWS, 2)

    gmat = jnp.asarray(np.kron(np.eye(_NH, dtype=np.float32),
                               np.ones((_NS, _NS), dtype=np.float32)))
    awT = aw_w.T
    awb = aw_b.reshape(1, _NH * _NS)
    soxT = so_w[0::2].T
    soyT = so_w[1::2].T
    sobx = so_b[0::2].reshape(1, _SAMP)
    soby = so_b[1::2].reshape(1, _SAMP)

    feats = [features_0, features_1, features_2, features_3]
    feats = [feats[i].reshape(b, feats[i].shape[1], _HW2[i])
             for i in range(_L)]
    epws = [ep_w0, ep_w1, ep_w2, ep_w3]
    idx, w_all, table = _fused_tc(x2d, ref2d, awT, awb, soxT, sobx,
                                  soyT, soby, gmat, feats, epws)

    out2d = _sc_gather(table, idx.reshape(-1), w_all)

    epbs = [ep_b0, ep_b1, ep_b2, ep_b3]
    bias = jnp.stack([jnp.tile(eb, _NH) for eb in epbs], axis=0)
    out = out2d.reshape(b, l, p, _NH * _HEAD_DIM) + bias[None, :, None, :]
    return out
